# baseline (device time: 212046 ns/iter reference)
import jax
import jax.numpy as jnp
from jax import lax
from jax.experimental import pallas as pl
from jax.experimental.pallas import tpu as pltpu

N_DEV = 8
SQ = 2048
SKV_LOCAL = 2048
HQ = 8
DH = 128
DM = HQ * DH
QBLK = 256
CBLK = 128
NBLK = SQ // QBLK
NCH = SQ // CBLK
SCALE = 0.08838834764831843

TREE_CHILDREN = (
    {0: (1,), 1: (2, 5), 2: (3, 6), 5: (4,), 6: (7,)},
    {0: (3,), 3: (2, 7), 2: (1,), 7: (6, 4), 6: (5,)},
    {0: (4,), 4: (5, 7), 5: (1, 6), 6: (2,), 7: (3,)},
)
TREE_PARENT = (
    {1: 0, 2: 1, 3: 2, 4: 5, 5: 1, 6: 2, 7: 6},
    {1: 2, 2: 3, 3: 0, 4: 7, 5: 6, 6: 7, 7: 3},
    {1: 5, 2: 6, 3: 7, 4: 0, 5: 4, 6: 5, 7: 4},
)
PEERS = {
    0: (1, 3, 4), 1: (0, 2, 5), 2: (1, 3, 6), 3: (0, 2, 7),
    4: (0, 5, 7), 5: (1, 4, 6), 6: (2, 5, 7), 7: (3, 4, 6),
}


def kernel(x, Wq, K_ext, V_ext, Wo):
    x2 = x[0]
    k2 = K_ext[0].reshape(SKV_LOCAL, DM).astype(jnp.bfloat16)
    v2 = V_ext[0].reshape(SKV_LOCAL, DM).astype(jnp.bfloat16)

    def body(x_ref, wq_ref, k_ref, v_ref, wo_ref, out_ref,
             ctx_ref, wqbf_ref, wobf_ref, send_sems, recv_sems):
        my = lax.axis_index("i")

        wobf_ref[...] = wo_ref[...].astype(jnp.bfloat16)

        def chunk_copy(c, target, d):
            return pltpu.make_async_remote_copy(
                src_ref=ctx_ref.at[pl.ds(c * CBLK, CBLK)],
                dst_ref=ctx_ref.at[pl.ds(c * CBLK, CBLK)],
                send_sem=send_sems.at[c, d],
                recv_sem=recv_sems.at[c],
                device_id=(target,), device_id_type=pl.DeviceIdType.MESH,
            )

        def project(c, rows):
            out_ref[c * rows:(c + 1) * rows, :] = jnp.dot(
                ctx_ref[c * rows:(c + 1) * rows, :], wobf_ref[...],
                preferred_element_type=jnp.float32,
            ).astype(jnp.bfloat16)

        def barrier(peers):
            bsem = pltpu.get_barrier_semaphore()
            for pr in peers:
                pl.semaphore_signal(
                    bsem, inc=1,
                    device_id=(pr,), device_id_type=pl.DeviceIdType.MESH,
                )
            pl.semaphore_wait(bsem, len(peers))

        @pl.when(my == 0)
        def _():
            barrier(PEERS[0])
            wqbf_ref[...] = wq_ref[...].astype(jnp.bfloat16)
            for b in range(NBLK):
                kmax = (b + 1) * QBLK
                xb = x_ref[b * QBLK:(b + 1) * QBLK, :].astype(jnp.bfloat16)
                qb = jnp.dot(
                    xb, wqbf_ref[...], preferred_element_type=jnp.float32
                ).astype(jnp.bfloat16)
                rows = b * QBLK + lax.broadcasted_iota(
                    jnp.int32, (QBLK, kmax), 0)
                cols = lax.broadcasted_iota(jnp.int32, (QBLK, kmax), 1)
                mask = (cols // 64) <= (rows // 64)
                for h in range(HQ):
                    kh = k_ref[:kmax, h * DH:(h + 1) * DH]
                    vh = v_ref[:kmax, h * DH:(h + 1) * DH]
                    qh = qb[:, h * DH:(h + 1) * DH]
                    s = lax.dot_general(
                        qh, kh, (((1,), (1,)), ((), ())),
                        preferred_element_type=jnp.float32,
                    ) * SCALE
                    s = jnp.where(mask, s, -1e9)
                    p32 = jnp.exp(s)
                    l = jnp.sum(p32, axis=-1, keepdims=True)
                    cun = jnp.dot(
                        p32.astype(jnp.bfloat16), vh,
                        preferred_element_type=jnp.float32,
                    )
                    ctx_ref[b * QBLK:(b + 1) * QBLK, h * DH:(h + 1) * DH] = (
                        (cun * (1.0 / l)).astype(jnp.bfloat16)
                    )
                for c in (2 * b, 2 * b + 1):
                    for d, tgt in enumerate(TREE_CHILDREN[c % 3][0]):
                        chunk_copy(c, tgt, d).start()
            for b in range(NBLK):
                project(b, QBLK)
            for c in range(NCH):
                for d, tgt in enumerate(TREE_CHILDREN[c % 3][0]):
                    chunk_copy(c, tgt, d).wait_send()

        for pos in range(1, N_DEV):

            @pl.when(my == pos)
            def _(pos=pos):
                barrier(PEERS[pos])
                for c in range(NCH):
                    t = c % 3
                    chunk_copy(c, TREE_PARENT[t][pos], 0).wait_recv()
                    for d, tgt in enumerate(TREE_CHILDREN[t].get(pos, ())):
                        chunk_copy(c, tgt, d).start()
                    project(c, CBLK)
                for c in range(NCH):
                    for d, tgt in enumerate(
                            TREE_CHILDREN[c % 3].get(pos, ())):
                        chunk_copy(c, tgt, d).wait_send()

    out = pl.pallas_call(
        body,
        out_shape=jax.ShapeDtypeStruct((SQ, DM), jnp.bfloat16),
        in_specs=[pl.BlockSpec(memory_space=pltpu.VMEM)] * 5,
        out_specs=pl.BlockSpec(memory_space=pltpu.VMEM),
        scratch_shapes=[
            pltpu.VMEM((SQ, DM), jnp.bfloat16),
            pltpu.VMEM((DM, DM), jnp.bfloat16),
            pltpu.VMEM((DM, DM), jnp.bfloat16),
            pltpu.SemaphoreType.DMA((NCH, 2)),
            pltpu.SemaphoreType.DMA((NCH,)),
        ],
        compiler_params=pltpu.CompilerParams(
            collective_id=0, vmem_limit_bytes=64 * 1024 * 1024
        ),
    )(x2, Wq, k2, v2, Wo)
    return out.reshape(1, SQ, DM)


# device time: 108578 ns/iter; 1.9529x vs baseline; 1.9529x over previous
import jax
import jax.numpy as jnp
from jax import lax
from jax.experimental import pallas as pl
from jax.experimental.pallas import tpu as pltpu

N_DEV = 8
SQ = 2048
SKV_LOCAL = 2048
HQ = 8
DH = 128
DM = HQ * DH
QBLK = 256
CBLK = 128
NBLK = SQ // QBLK
NCH = SQ // CBLK
SCALE = 0.08838834764831843

TREE_CHILDREN = (
    {0: (1,), 1: (2, 5), 2: (3, 6), 5: (4,), 6: (7,)},
    {0: (3,), 3: (2, 7), 2: (1,), 7: (6, 4), 6: (5,)},
    {0: (4,), 4: (5, 7), 5: (1, 6), 6: (2,), 7: (3,)},
)
TREE_PARENT = (
    {1: 0, 2: 1, 3: 2, 4: 5, 5: 1, 6: 2, 7: 6},
    {1: 2, 2: 3, 3: 0, 4: 7, 5: 6, 6: 7, 7: 3},
    {1: 5, 2: 6, 3: 7, 4: 0, 5: 4, 6: 5, 7: 4},
)
PEERS = {
    0: (1, 3, 4), 1: (0, 2, 5), 2: (1, 3, 6), 3: (0, 2, 7),
    4: (0, 5, 7), 5: (1, 4, 6), 6: (2, 5, 7), 7: (3, 4, 6),
}
DEPTH = {
    1: (1, 3, 3), 2: (2, 2, 4), 3: (3, 1, 3), 4: (3, 3, 1),
    5: (2, 4, 2), 6: (3, 3, 3), 7: (4, 2, 2),
}


def _wait_order(pos):
    def key(c):
        b = c // 2
        t_prod = 1.5 * (b + 1) + 0.5 * (b + 1) ** 2
        return (t_prod + 5.0 * DEPTH[pos][c % 3], c)
    return sorted(range(NCH), key=key)


def kernel(x, Wq, K_ext, V_ext, Wo):
    x2 = x[0]
    k2 = K_ext[0].reshape(SKV_LOCAL, DM).astype(jnp.bfloat16)
    v2 = V_ext[0].reshape(SKV_LOCAL, DM).astype(jnp.bfloat16)

    def body(x_ref, wq_ref, k_ref, v_ref, wo_ref, out_ref,
             ctx_ref, wqbf_ref, wobf_ref, send_sems, recv_sems):
        my = lax.axis_index("i")

        wobf_ref[...] = wo_ref[...].astype(jnp.bfloat16)

        def chunk_copy(c, target, d):
            return pltpu.make_async_remote_copy(
                src_ref=ctx_ref.at[pl.ds(c * CBLK, CBLK)],
                dst_ref=ctx_ref.at[pl.ds(c * CBLK, CBLK)],
                send_sem=send_sems.at[c, d],
                recv_sem=recv_sems.at[c],
                device_id=(target,), device_id_type=pl.DeviceIdType.MESH,
            )

        def project(c, rows):
            out_ref[c * rows:(c + 1) * rows, :] = jnp.dot(
                ctx_ref[c * rows:(c + 1) * rows, :], wobf_ref[...],
                preferred_element_type=jnp.float32,
            ).astype(jnp.bfloat16)

        def barrier(peers):
            bsem = pltpu.get_barrier_semaphore()
            for pr in peers:
                pl.semaphore_signal(
                    bsem, inc=1,
                    device_id=(pr,), device_id_type=pl.DeviceIdType.MESH,
                )
            pl.semaphore_wait(bsem, len(peers))

        @pl.when(my == 0)
        def _():
            barrier(PEERS[0])
            wqbf_ref[...] = wq_ref[...].astype(jnp.bfloat16)
            for b in range(NBLK):
                kmax = (b + 1) * QBLK
                xb = x_ref[b * QBLK:(b + 1) * QBLK, :].astype(jnp.bfloat16)
                qb = jnp.dot(
                    xb, wqbf_ref[...], preferred_element_type=jnp.float32
                ).astype(jnp.bfloat16)
                rows = b * QBLK + lax.broadcasted_iota(
                    jnp.int32, (QBLK, kmax), 0)
                cols = lax.broadcasted_iota(jnp.int32, (QBLK, kmax), 1)
                mask = (cols // 64) <= (rows // 64)
                for h in range(HQ):
                    kh = k_ref[:kmax, h * DH:(h + 1) * DH]
                    vh = v_ref[:kmax, h * DH:(h + 1) * DH]
                    qh = qb[:, h * DH:(h + 1) * DH]
                    s = lax.dot_general(
                        qh, kh, (((1,), (1,)), ((), ())),
                        preferred_element_type=jnp.float32,
                    ) * SCALE
                    s = jnp.where(mask, s, -1e9)
                    p32 = jnp.exp(s)
                    l = jnp.sum(p32, axis=-1, keepdims=True)
                    cun = jnp.dot(
                        p32.astype(jnp.bfloat16), vh,
                        preferred_element_type=jnp.float32,
                    )
                    ctx_ref[b * QBLK:(b + 1) * QBLK, h * DH:(h + 1) * DH] = (
                        (cun * (1.0 / l)).astype(jnp.bfloat16)
                    )
                for c in (2 * b, 2 * b + 1):
                    for d, tgt in enumerate(TREE_CHILDREN[c % 3][0]):
                        chunk_copy(c, tgt, d).start()
            for b in range(NBLK):
                project(b, QBLK)
            for c in range(NCH):
                for d, tgt in enumerate(TREE_CHILDREN[c % 3][0]):
                    chunk_copy(c, tgt, d).wait_send()

        for pos in range(1, N_DEV):

            @pl.when(my == pos)
            def _(pos=pos):
                barrier(PEERS[pos])
                for c in _wait_order(pos):
                    t = c % 3
                    chunk_copy(c, TREE_PARENT[t][pos], 0).wait_recv()
                    for d, tgt in enumerate(TREE_CHILDREN[t].get(pos, ())):
                        chunk_copy(c, tgt, d).start()
                    project(c, CBLK)
                for c in range(NCH):
                    for d, tgt in enumerate(
                            TREE_CHILDREN[c % 3].get(pos, ())):
                        chunk_copy(c, tgt, d).wait_send()

    out = pl.pallas_call(
        body,
        out_shape=jax.ShapeDtypeStruct((SQ, DM), jnp.bfloat16),
        in_specs=[pl.BlockSpec(memory_space=pltpu.VMEM)] * 5,
        out_specs=pl.BlockSpec(memory_space=pltpu.VMEM),
        scratch_shapes=[
            pltpu.VMEM((SQ, DM), jnp.bfloat16),
            pltpu.VMEM((DM, DM), jnp.bfloat16),
            pltpu.VMEM((DM, DM), jnp.bfloat16),
            pltpu.SemaphoreType.DMA((NCH, 2)),
            pltpu.SemaphoreType.DMA((NCH,)),
        ],
        compiler_params=pltpu.CompilerParams(
            collective_id=0, vmem_limit_bytes=64 * 1024 * 1024
        ),
    )(x2, Wq, k2, v2, Wo)
    return out.reshape(1, SQ, DM)
